# baseline (speedup 1.0000x reference)
import functools

import jax
import jax.numpy as jnp
from jax import lax
from jax.experimental import pallas as pl
from jax.experimental.pallas import tpu as pltpu
from jax.experimental.pallas import tpu_sc as plsc

_NBUF = 4


@functools.cache
def _build_sc_kernel(B, L, V, D):
    info = plsc.get_sparse_core_info()
    NW = info.num_cores * info.num_subcores
    NLANE = info.num_lanes
    B_PER = B // NW
    assert L <= 128 and D % NLANE == 0 and B_PER % _NBUF == 0
    n_vec = D // NLANE

    mesh = plsc.VectorSubcoreMesh(core_axis_name="c", subcore_axis_name="s")

    @functools.partial(
        pl.kernel,
        mesh=mesh,
        out_type=jax.ShapeDtypeStruct((NW, B_PER * D), jnp.float32),
        scratch_types=[
            pltpu.VMEM((B_PER, L), jnp.int32),
            pltpu.VMEM((_NBUF, L, D), jnp.float32),
            pltpu.VMEM((B_PER * D,), jnp.float32),
            [pltpu.SemaphoreType.DMA] * _NBUF,
        ],
        compiler_params=pltpu.CompilerParams(use_tc_tiling_on_sc=False),
    )
    def body(idx_hbm, table_hbm, out_hbm, idx_v, bufs, out_v, sems):
        wid = lax.axis_index("s") * info.num_cores + lax.axis_index("c")
        pltpu.sync_copy(idx_hbm.at[wid], idx_v)

        def start(c, b):
            pltpu.async_copy(table_hbm.at[idx_v.at[c]], bufs.at[b], sems[b])

        def drain(c, b):
            pltpu.make_async_copy(
                table_hbm.at[idx_v.at[c]], bufs.at[b], sems[b]
            ).wait()

        def reduce_row(c, b):
            accs = [bufs[b, 0, pl.ds(v * NLANE, NLANE)] for v in range(n_vec)]
            for j in range(1, L):
                for v in range(n_vec):
                    accs[v] += bufs[b, j, pl.ds(v * NLANE, NLANE)]
            off = c * D
            for v in range(n_vec):
                out_v[pl.ds(off + v * NLANE, NLANE)] = accs[v] * (1.0 / L)

        for b in range(_NBUF):
            start(b, b)

        def loop_body(g, carry):
            for b in range(_NBUF):
                c = g * _NBUF + b
                drain(c, b)
                reduce_row(c, b)

                @pl.when(g < B_PER // _NBUF - 1)
                def _():
                    start(c + _NBUF, b)
            return carry

        lax.fori_loop(0, B_PER // _NBUF, loop_body, 0)
        pltpu.sync_copy(out_v, out_hbm.at[wid])

    return body


def kernel(x, W):
    B, L = x.shape
    V, D = W.shape
    NW = 32
    sc = _build_sc_kernel(B, L, V, D)
    idx = x.astype(jnp.int32).reshape(NW, B // NW, L)
    out = sc(idx, W)
    return out.reshape(B, D)


# speedup vs baseline: 1.0405x; 1.0405x over previous
import functools

import jax
import jax.numpy as jnp
from jax import lax
from jax.experimental import pallas as pl
from jax.experimental.pallas import tpu as pltpu
from jax.experimental.pallas import tpu_sc as plsc

_NBUF = 4


@functools.cache
def _build_sc_kernel(B, L, V, D):
    info = plsc.get_sparse_core_info()
    NW = info.num_cores * info.num_subcores
    NL = info.num_lanes
    B_PER = B // NW
    GIDX = 2 * L
    NGRP = B_PER // 2
    LC = -(-L // NL)
    assert GIDX <= 128 and D % NL == 0 and NGRP % _NBUF == 0
    n_vec = D // NL

    mesh = plsc.VectorSubcoreMesh(core_axis_name="c", subcore_axis_name="s")

    @functools.partial(
        pl.kernel,
        mesh=mesh,
        out_type=jax.ShapeDtypeStruct((D, B), jnp.float32),
        scratch_types=[
            pltpu.VMEM((LC * NL, B_PER), jnp.int32),
            pltpu.VMEM((NGRP, GIDX), jnp.int32),
            pltpu.VMEM((_NBUF, GIDX, D), jnp.float32),
            pltpu.VMEM((D, B_PER), jnp.float32),
            [pltpu.SemaphoreType.DMA] * _NBUF,
        ],
        compiler_params=pltpu.CompilerParams(use_tc_tiling_on_sc=False,
                                             needs_layout_passes=False),
    )
    def body(xt_hbm, table_hbm, out_hbm, stage, idx_v, bufs, out_t, sems):
        wid = lax.axis_index("s") * info.num_cores + lax.axis_index("c")
        col0 = wid * B_PER
        pltpu.sync_copy(xt_hbm.at[:, pl.ds(col0, B_PER)],
                        stage.at[pl.ds(0, L)])

        iota = lax.iota(jnp.int32, NL)

        def transpose_body(b, carry):
            row = jnp.full((NL,), b // 2, jnp.int32)
            cbase = (b % 2) * L
            for jc in range(LC):
                j = jc * NL + iota
                vals = plsc.load_gather(
                    stage, [j, jnp.full((NL,), b, jnp.int32)])
                if (jc + 1) * NL <= L:
                    plsc.store_scatter(idx_v, [row, cbase + j], vals)
                else:
                    plsc.store_scatter(idx_v, [row, cbase + j], vals,
                                       mask=j < L)
            return carry

        lax.fori_loop(0, B_PER, transpose_body, 0)

        def start(c, b):
            pltpu.async_copy(table_hbm.at[idx_v.at[c]], bufs.at[b], sems[b])

        def drain(c, b):
            pltpu.make_async_copy(
                table_hbm.at[idx_v.at[c]], bufs.at[b], sems[b]
            ).wait()

        def reduce_group(c, b):
            for k in range(2):
                base = k * L
                accs = [bufs[b, base, pl.ds(v * NL, NL)]
                        for v in range(n_vec)]
                for j in range(1, L):
                    for v in range(n_vec):
                        accs[v] += bufs[b, base + j, pl.ds(v * NL, NL)]
                r = jnp.full((NL,), c * 2 + k, jnp.int32)
                for v in range(n_vec):
                    plsc.store_scatter(out_t, [v * NL + iota, r],
                                       accs[v] * (1.0 / L))

        for b in range(_NBUF):
            start(b, b)

        def loop_body(g, carry):
            for b in range(_NBUF):
                c = g * _NBUF + b
                drain(c, b)
                reduce_group(c, b)

                @pl.when(g < NGRP // _NBUF - 1)
                def _():
                    start(c + _NBUF, b)
            return carry

        lax.fori_loop(0, NGRP // _NBUF, loop_body, 0)
        pltpu.sync_copy(out_t, out_hbm.at[:, pl.ds(col0, B_PER)])

    return body


def kernel(x, W):
    B, L = x.shape
    V, D = W.shape
    sc = _build_sc_kernel(B, L, V, D)
    out_t = sc(x.T.astype(jnp.int32), W)
    return out_t.T
